# Initial kernel scaffold; baseline (speedup 1.0000x reference)
import functools

import jax
import jax.numpy as jnp
from jax import lax
from jax.experimental import pallas as pl
from jax.experimental.pallas import tpu as pltpu
from jax.experimental.pallas import tpu_sc as plsc

NUM_U = 5000
NUM_I = 5000
N = NUM_U + NUM_I
D = 128
E = 320000
NC = 2
NS = 16
NW = NC * NS
CHUNK = 128
CH_PER_W = 79
E_PER_W = CH_PER_W * CHUNK
E_PAD = E_PER_W * NW
ROWS_PER_TILE = N // NS


def _sc_propagate(ego, cols, rows, vals, zeros):
    mesh = plsc.VectorSubcoreMesh(
        core_axis_name="c", subcore_axis_name="s",
        num_cores=NC, num_subcores=NS)

    @functools.partial(
        pl.kernel,
        out_type=jax.ShapeDtypeStruct((NC * N, D), jnp.float32),
        mesh=mesh,
        scratch_types=[
            pltpu.VMEM((CHUNK,), jnp.int32),
            pltpu.VMEM((CHUNK,), jnp.int32),
            pltpu.VMEM((CHUNK,), jnp.float32),
            pltpu.VMEM((CHUNK, D), jnp.float32),
            pltpu.VMEM_SHARED((N, D), jnp.float32),
            pltpu.SemaphoreType.DMA,
        ],
    )
    def k(ego_h, cols_h, rows_h, vals_h, zeros_h, part_h,
          colb, rowb, valb, gathb, acc, sem):
        c = lax.axis_index("c")
        s = lax.axis_index("s")
        w = c * NS + s
        pltpu.sync_copy(zeros_h, acc.at[pl.ds(s * ROWS_PER_TILE, ROWS_PER_TILE)])
        plsc.subcore_barrier()

        ebase = w * E_PER_W

        def chunk(i, carry):
            base = ebase + i * CHUNK
            pltpu.sync_copy(cols_h.at[pl.ds(base, CHUNK)], colb)
            pltpu.sync_copy(rows_h.at[pl.ds(base, CHUNK)], rowb)
            pltpu.sync_copy(vals_h.at[pl.ds(base, CHUNK)], valb)
            pltpu.async_copy(ego_h.at[colb], gathb, sem).wait()

            def edge(e, carry2):
                v = valb[e]
                for j in range(D // 16):
                    sl = pl.ds(j * 16, 16)
                    gathb[e, sl] = gathb[e, sl] * v
                return carry2

            lax.fori_loop(0, CHUNK, edge, 0)
            pltpu.sync_copy(gathb, acc.at[rowb], add=True)
            return carry

        lax.fori_loop(0, CH_PER_W, chunk, 0)
        plsc.subcore_barrier()
        r0 = s * ROWS_PER_TILE
        pltpu.sync_copy(acc.at[pl.ds(r0, ROWS_PER_TILE)],
                        part_h.at[pl.ds(c * N + r0, ROWS_PER_TILE)])

    return k(ego, cols, rows, vals, zeros)


def _tc_merge(part):
    blk = 250

    def body(a_ref, b_ref, o_ref):
        o_ref[...] = a_ref[...] + b_ref[...]

    return pl.pallas_call(
        body,
        grid=(N // blk,),
        in_specs=[
            pl.BlockSpec((blk, D), lambda i: (i, 0)),
            pl.BlockSpec((blk, D), lambda i: (i + N // blk, 0)),
        ],
        out_specs=pl.BlockSpec((blk, D), lambda i: (i, 0)),
        out_shape=jax.ShapeDtypeStruct((N, D), jnp.float32),
    )(part, part)


def _tc_user_mean(u0, e1, e2, e3):
    blk = 250

    def body(a_ref, b_ref, c_ref, d_ref, o_ref):
        o_ref[...] = (a_ref[...] + b_ref[...] + c_ref[...] + d_ref[...]) * 0.25

    return pl.pallas_call(
        body,
        grid=(NUM_U // blk,),
        in_specs=[pl.BlockSpec((blk, D), lambda i: (i, 0))] * 4,
        out_specs=pl.BlockSpec((blk, D), lambda i: (i, 0)),
        out_shape=jax.ShapeDtypeStruct((NUM_U, D), jnp.float32),
    )(u0, e1, e2, e3)


def kernel(adj_indices, adj_values, user_emb, item_emb):
    row = adj_indices[0]
    col = adj_indices[1]
    pad = E_PAD - E
    rows = jnp.concatenate([row, jnp.zeros((pad,), jnp.int32)])
    cols = jnp.concatenate([col, jnp.zeros((pad,), jnp.int32)])
    vals = jnp.concatenate([adj_values, jnp.zeros((pad,), jnp.float32)])
    zeros = jnp.zeros((ROWS_PER_TILE, D), jnp.float32)

    ego0 = jnp.concatenate([user_emb, item_emb], axis=0)
    egos = [ego0]
    for _ in range(3):
        part = _sc_propagate(egos[-1], cols, rows, vals, zeros)
        egos.append(_tc_merge(part))

    user_out = _tc_user_mean(user_emb, egos[1], egos[2], egos[3])
    item_embs = (item_emb, egos[1][NUM_U:], egos[2][NUM_U:], egos[3][NUM_U:])
    return (user_out, item_embs)


# speedup vs baseline: 3.3498x; 3.3498x over previous
import functools

import jax
import jax.numpy as jnp
from jax import lax
from jax.experimental import pallas as pl
from jax.experimental.pallas import tpu as pltpu
from jax.experimental.pallas import tpu_sc as plsc

NUM_U = 5000
NUM_I = 5000
N = NUM_U + NUM_I
NP = 10240
D = 128
E = 320000
NC = 2
NS = 16
NW = NC * NS
CHUNK = 128
CH_PER_W = 79
E_PER_W = CH_PER_W * CHUNK
E_PAD = E_PER_W * NW
ROWS_PER_TILE = NP // NS


def _sc_propagate(ego, cols, rows, vals, zeros):
    mesh = plsc.VectorSubcoreMesh(
        core_axis_name="c", subcore_axis_name="s",
        num_cores=NC, num_subcores=NS)

    @functools.partial(
        pl.kernel,
        out_type=jax.ShapeDtypeStruct((NC * NP, D), jnp.float32),
        mesh=mesh,
        scratch_types=[
            pltpu.VMEM((CHUNK,), jnp.int32),
            pltpu.VMEM((CHUNK,), jnp.int32),
            pltpu.VMEM((CHUNK,), jnp.float32),
            pltpu.VMEM((CHUNK, D), jnp.float32),
            pltpu.VMEM_SHARED((NP, D), jnp.float32),
            pltpu.SemaphoreType.DMA,
        ],
    )
    def k(ego_h, cols_h, rows_h, vals_h, zeros_h, part_h,
          colb, rowb, valb, gathb, acc, sem):
        c = lax.axis_index("c")
        s = lax.axis_index("s")
        w = c * NS + s
        pltpu.sync_copy(zeros_h, acc.at[pl.ds(s * ROWS_PER_TILE, ROWS_PER_TILE)])
        plsc.subcore_barrier()

        ebase = w * E_PER_W

        def chunk(i, carry):
            base = ebase + i * CHUNK
            pltpu.sync_copy(cols_h.at[pl.ds(base, CHUNK)], colb)
            pltpu.sync_copy(rows_h.at[pl.ds(base, CHUNK)], rowb)
            pltpu.sync_copy(vals_h.at[pl.ds(base, CHUNK)], valb)
            pltpu.async_copy(ego_h.at[colb], gathb, sem).wait()

            def group(g, carry2):
                vals16 = valb[pl.ds(g * 16, 16)]
                for k in range(16):
                    v = vals16[k]
                    e = g * 16 + k
                    for j in range(D // 16):
                        sl = pl.ds(j * 16, 16)
                        gathb[e, sl] = gathb[e, sl] * v
                return carry2

            lax.fori_loop(0, CHUNK // 16, group, 0)
            pltpu.sync_copy(gathb, acc.at[rowb], add=True)
            return carry

        lax.fori_loop(0, CH_PER_W, chunk, 0)
        plsc.subcore_barrier()
        r0 = s * ROWS_PER_TILE
        pltpu.sync_copy(acc.at[pl.ds(r0, ROWS_PER_TILE)],
                        part_h.at[pl.ds(c * NP + r0, ROWS_PER_TILE)])

    return k(ego, cols, rows, vals, zeros)


def _tc_merge(part):
    blk = 640

    def body(a_ref, b_ref, o_ref):
        o_ref[...] = a_ref[...] + b_ref[...]

    return pl.pallas_call(
        body,
        grid=(NP // blk,),
        in_specs=[
            pl.BlockSpec((blk, D), lambda i: (i, 0)),
            pl.BlockSpec((blk, D), lambda i: (i + NP // blk, 0)),
        ],
        out_specs=pl.BlockSpec((blk, D), lambda i: (i, 0)),
        out_shape=jax.ShapeDtypeStruct((NP, D), jnp.float32),
    )(part, part)


def _tc_user_mean(u0, e1, e2, e3):
    blk = 200

    def body(a_ref, b_ref, c_ref, d_ref, o_ref):
        o_ref[...] = (a_ref[...] + b_ref[...] + c_ref[...] + d_ref[...]) * 0.25

    return pl.pallas_call(
        body,
        grid=(NUM_U // blk,),
        in_specs=[pl.BlockSpec((blk, D), lambda i: (i, 0))] * 4,
        out_specs=pl.BlockSpec((blk, D), lambda i: (i, 0)),
        out_shape=jax.ShapeDtypeStruct((NUM_U, D), jnp.float32),
    )(u0, e1, e2, e3)


def kernel(adj_indices, adj_values, user_emb, item_emb):
    row = adj_indices[0]
    col = adj_indices[1]
    pad = E_PAD - E
    rows = jnp.concatenate([row, jnp.zeros((pad,), jnp.int32)])
    cols = jnp.concatenate([col, jnp.zeros((pad,), jnp.int32)])
    vals = jnp.concatenate([adj_values, jnp.zeros((pad,), jnp.float32)])
    zeros = jnp.zeros((ROWS_PER_TILE, D), jnp.float32)

    ego0 = jnp.concatenate(
        [user_emb, item_emb, jnp.zeros((NP - N, D), jnp.float32)], axis=0)
    egos = [ego0]
    for _ in range(3):
        part = _sc_propagate(egos[-1], cols, rows, vals, zeros)
        egos.append(_tc_merge(part))

    user_out = _tc_user_mean(user_emb, egos[1], egos[2], egos[3])
    item_embs = (item_emb, egos[1][NUM_U:N], egos[2][NUM_U:N], egos[3][NUM_U:N])
    return (user_out, item_embs)
